# baseline (speedup 1.0000x reference)
import functools

import jax
import jax.numpy as jnp
from jax import lax
from jax.experimental import pallas as pl
from jax.experimental.pallas import tpu as pltpu
from jax.experimental.pallas import tpu_sc as plsc

D = 32
B = 4096
K = 20
NUM_ROWS = 1000000
RC = 65536
RG = 16
PAD_ROWS = RG * RC
NC = 2
NS = 16
NW = NC * NS
E_W = B // NW


def _sc_gather_body(table, pos_src, pos_dst, neg_src, neg_dst,
                    o_ps, o_pd, o_ns, o_nd,
                    idx_v, pidx_v, rows_a, rows_b, prow_v, acc_v,
                    sem_p, sem_a, sem_b):
    wid = lax.axis_index("s") * NC + lax.axis_index("c")
    base = wid * E_W

    mask_hi = jnp.int32(-65536)

    def unpack(v):
        h = plsc.bitcast(v & mask_hi, jnp.float32)
        l = plsc.bitcast(v << 16, jnp.float32)
        return h, l

    pltpu.sync_copy(pos_src.at[pl.ds(base, E_W)], pidx_v.at[0])
    pltpu.sync_copy(pos_dst.at[pl.ds(base, E_W)], pidx_v.at[1])
    pltpu.sync_copy(neg_src.at[wid], idx_v.at[pl.ds(0, K)])
    pltpu.sync_copy(neg_dst.at[wid], idx_v.at[pl.ds(K, K)])

    cp_p1 = pltpu.async_copy(table.at[pidx_v.at[0]],
                             prow_v.at[pl.ds(0, E_W)], sem_p)
    cp_p2 = pltpu.async_copy(table.at[pidx_v.at[1]],
                             prow_v.at[pl.ds(E_W, E_W)], sem_p)
    cps_a = [pltpu.async_copy(table.at[idx_v.at[c]],
                              rows_a.at[pl.ds(c * E_W, E_W)], sem_a)
             for c in range(K)]
    cps_b = [pltpu.async_copy(table.at[idx_v.at[K + c]],
                              rows_b.at[pl.ds(c * E_W, E_W)], sem_b)
             for c in range(K)]

    cp_p1.wait()
    cp_p2.wait()

    def pos_out(off, out_hbm):
        def unpack_row(e, carry):
            h, l = unpack(prow_v[off + e, :])
            acc_v[e, pl.ds(0, 16)] = h
            acc_v[e, pl.ds(16, 16)] = l
            return carry

        lax.fori_loop(0, E_W, unpack_row, 0)
        pltpu.sync_copy(acc_v, out_hbm.at[pl.ds(base, E_W)])

    pos_out(0, o_ps)
    pos_out(E_W, o_pd)

    def neg_out(rows_v, out_hbm):
        def acc_edge(e, carry):
            r = e * K
            a0, a1 = unpack(rows_v[r, :])
            for kk in range(1, K):
                h, l = unpack(rows_v[r + kk, :])
                a0 = a0 + h
                a1 = a1 + l
            acc_v[e, pl.ds(0, 16)] = a0
            acc_v[e, pl.ds(16, 16)] = a1
            return carry

        lax.fori_loop(0, E_W, acc_edge, 0)
        pltpu.sync_copy(acc_v, out_hbm.at[pl.ds(base, E_W)])

    for cp in cps_a:
        cp.wait()
    neg_out(rows_a, o_ns)
    for cp in cps_b:
        cp.wait()
    neg_out(rows_b, o_nd)


_sc_gather = functools.partial(
    pl.kernel,
    mesh=plsc.VectorSubcoreMesh(core_axis_name="c", subcore_axis_name="s"),
    out_type=[jax.ShapeDtypeStruct((B, D), jnp.float32)] * 4,
    scratch_types=[
        pltpu.VMEM((2 * K, E_W), jnp.int32),
        pltpu.VMEM((2, E_W), jnp.int32),
        pltpu.VMEM((K * E_W, 16), jnp.int32),
        pltpu.VMEM((K * E_W, 16), jnp.int32),
        pltpu.VMEM((2 * E_W, 16), jnp.int32),
        pltpu.VMEM((E_W, D), jnp.float32),
        pltpu.SemaphoreType.DMA,
        pltpu.SemaphoreType.DMA,
        pltpu.SemaphoreType.DMA,
    ],
    compiler_params=pltpu.CompilerParams(use_tc_tiling_on_sc=False,
                                         needs_layout_passes=False),
)(_sc_gather_body)


def _repack_body(at_ref, ehi_ref, elo_ref, out_ref):
    i = pl.program_id(0)
    x = at_ref[...]
    cols = i * RC + jax.lax.broadcasted_iota(jnp.int32, (32, RC), 1)
    x = jnp.where(cols < NUM_ROWS, x, 0.0)
    s = RC // 8
    xc = jnp.concatenate([x[:, j * s:(j + 1) * s] for j in range(8)],
                         axis=0)
    dims = (((0,), (0,)), ((), ()))
    hi = jax.lax.dot_general(xc, ehi_ref[...], dims,
                             preferred_element_type=jnp.float32)
    lo = jax.lax.dot_general(xc, elo_ref[...], dims,
                             preferred_element_type=jnp.float32)
    hi_i = jax.lax.bitcast_convert_type(hi, jnp.int32)
    lo_i = jax.lax.bitcast_convert_type(lo, jnp.int32)
    hi_b = (hi_i + 32768) & jnp.int32(-65536)
    lo_b = jax.lax.shift_right_logical(lo_i + 32768, 16)
    out_ref[...] = hi_b | lo_b


_repack = pl.pallas_call(
    _repack_body,
    grid=(RG,),
    in_specs=[pl.BlockSpec((32, RC), lambda i: (0, i)),
              pl.BlockSpec((256, 128), lambda i: (0, 0)),
              pl.BlockSpec((256, 128), lambda i: (0, 0))],
    out_specs=pl.BlockSpec((RC // 8, 128), lambda i: (i, 0)),
    out_shape=jax.ShapeDtypeStruct((PAD_ROWS // 8, 128), jnp.int32),
)


_QSHIFT = (RC // 8).bit_length() - 1


def _permute_idx(r):
    return ((r & ~(RC - 1)) + ((r & (RC // 8 - 1)) << 3)
            + ((r & (RC - 1)) >> _QSHIFT))


def _selection_mats():
    p = jnp.arange(256, dtype=jnp.int32)[:, None]
    c = jnp.arange(128, dtype=jnp.int32)[None, :]
    base = (c // 16) * 32 + (c % 16) * 2
    ehi = (p == base).astype(jnp.float32)
    elo = (p == base + 1).astype(jnp.float32)
    return ehi, elo


def _tc_loss_body(ps_ref, pd_ref, ns_ref, nd_ref, w_ref, out_ref):
    eps = 1e-8
    epsilon = 1e-7
    w = w_ref[...]

    def norm(x):
        rms = jnp.sqrt(jnp.sum(x * x, axis=-1, keepdims=True) * (1.0 / D))
        return x / (rms + eps) * w

    a = norm(ps_ref[...])
    b = norm(pd_ref[...])
    c = norm(ns_ref[...] * (1.0 / K))
    d = norm(nd_ref[...] * (1.0 / K))
    dpos = jnp.sum(a * b, axis=-1, keepdims=True)
    dneg = jnp.sum(c * d, axis=-1, keepdims=True)

    def log_sig(x):
        return jnp.log(1.0 / (1.0 + jnp.exp(-x)) + epsilon)

    def log_one_minus_sig(x):
        return jnp.log(1.0 - 1.0 / (1.0 + jnp.exp(-x)) + epsilon)

    one = log_sig(dpos * (1.0 / D)) + log_one_minus_sig(dneg * (1.0 / D))
    two = log_sig(dpos) + log_one_minus_sig(dneg)
    out_ref[0, 0] = -(jnp.sum(one) + jnp.sum(two)) * (1.0 / B)


_tc_loss = pl.pallas_call(
    _tc_loss_body,
    out_shape=jax.ShapeDtypeStruct((1, 1), jnp.float32),
    out_specs=pl.BlockSpec(memory_space=pltpu.SMEM),
)


def kernel(pos_edges, neg_edges, node_embeddings, rms_weight):
    pos_src = _permute_idx(pos_edges[0])
    pos_dst = _permute_idx(pos_edges[1])
    neg_src = _permute_idx(neg_edges[0]).reshape(NW, K, E_W)
    neg_dst = _permute_idx(neg_edges[1]).reshape(NW, K, E_W)
    ehi, elo = _selection_mats()
    table_pk = _repack(node_embeddings.T, ehi, elo).reshape(PAD_ROWS, 16)
    ps, pd, ns, nd = _sc_gather(table_pk, pos_src, pos_dst,
                                neg_src, neg_dst)
    w_perm = jnp.concatenate([rms_weight[0::2], rms_weight[1::2]])
    loss = _tc_loss(ps, pd, ns, nd, w_perm.reshape(1, D))
    return loss[0, 0]
